# baseline (speedup 1.0000x reference)
import functools

import jax
import jax.numpy as jnp
from jax import lax
from jax.experimental import pallas as pl
from jax.experimental.pallas import tpu as pltpu
from jax.experimental.pallas import tpu_sc as plsc

NC = 2
NS = 16
NW = NC * NS
L = 16

E_TOTAL = 320000
W_EDGES = E_TOTAL // NW
CHUNK = 128
N_FULL = W_EDGES // CHUNK
TAIL = W_EDGES - N_FULL * CHUNK
D = 128
DW = D // 2
V = 10000
ROWS_PER_SUB = V // NS
BLK = 125


def _edge_dot_sigmoid(z, e):
    mesh = plsc.VectorSubcoreMesh(core_axis_name="c", subcore_axis_name="s")

    @functools.partial(
        pl.kernel,
        mesh=mesh,
        out_type=jax.ShapeDtypeStruct((E_TOTAL,), jnp.float32),
        compiler_params=pltpu.CompilerParams(
            needs_layout_passes=False, use_tc_tiling_on_sc=False),
        scratch_types=[
            pltpu.VMEM((W_EDGES,), jnp.int32),
            pltpu.VMEM((W_EDGES,), jnp.int32),
            pltpu.VMEM((2, CHUNK, DW), jnp.int32),
            pltpu.VMEM((2, CHUNK, DW), jnp.int32),
            pltpu.VMEM((CHUNK * L,), jnp.float32),
            pltpu.VMEM((W_EDGES,), jnp.float32),
            pltpu.VMEM((BLK, D), jnp.float32),
            pltpu.VMEM((BLK, DW), jnp.int32),
            pltpu.VMEM_SHARED((V, DW), jnp.int32),
            pltpu.SemaphoreType.DMA,
            pltpu.SemaphoreType.DMA,
            pltpu.SemaphoreType.DMA,
            pltpu.SemaphoreType.DMA,
        ],
    )
    def k(z_hbm, e_hbm, out_hbm,
          idx0, idx1, rows0, rows1, tmp, out_v, stage_f, pack_b, z_sh,
          s0a, s1a, s0b, s1b):
        wid = lax.axis_index("s") * NC + lax.axis_index("c")
        sid = lax.axis_index("s")

        pltpu.sync_copy(e_hbm.at[0, pl.ds(wid * W_EDGES, W_EDGES)], idx0)
        pltpu.sync_copy(e_hbm.at[1, pl.ds(wid * W_EDGES, W_EDGES)], idx1)
        r0s = sid * ROWS_PER_SUB

        def stage_block(blk, carry):
            row0 = r0s + blk * BLK
            pltpu.sync_copy(z_hbm.at[pl.ds(row0, BLK)], stage_f)

            @plsc.parallel_loop(0, BLK, unroll=2)
            def pack_row(r):
                for w in range(D // (2 * L)):
                    a = stage_f[r, pl.ds(w * 2 * L, L)]
                    b = stage_f[r, pl.ds(w * 2 * L + L, L)]
                    pk = plsc.pack(a, b, format=plsc.PackFormat.INTERLEAVED)
                    pack_b[r, pl.ds(w * L, L)] = plsc.bitcast(pk, jnp.int32)

            pltpu.sync_copy(pack_b, z_sh.at[pl.ds(row0, BLK)])
            return carry

        lax.fori_loop(0, ROWS_PER_SUB // BLK, stage_block, 0, unroll=False)
        plsc.subcore_barrier()

        z_sh_i = z_sh

        def start(c, n, buf, sem0, sem1):
            pltpu.async_copy(z_sh_i.at[idx0.at[pl.ds(c * CHUNK, n)]],
                             rows0.at[buf, pl.ds(0, n)], sem0)
            pltpu.async_copy(z_sh_i.at[idx1.at[pl.ds(c * CHUNK, n)]],
                             rows1.at[buf, pl.ds(0, n)], sem1)

        def wait(c, n, buf, sem0, sem1):
            pltpu.make_async_copy(
                z_sh_i.at[idx0.at[pl.ds(c * CHUNK, n)]],
                rows0.at[buf, pl.ds(0, n)], sem0).wait()
            pltpu.make_async_copy(
                z_sh_i.at[idx1.at[pl.ds(c * CHUNK, n)]],
                rows1.at[buf, pl.ds(0, n)], sem1).wait()

        def compute(c, n, buf):
            r0 = rows0.at[buf]
            r1 = rows1.at[buf]
            hi_mask = jnp.full((L,), -65536, jnp.int32)
            lane15 = lax.iota(jnp.int32, L) * L + (L - 1)

            @plsc.parallel_loop(0, n // L, unroll=2 if n > L else 1)
            def group_body(g):
                row = g * L
                base = g * (L * L)
                for edge in range(L):
                    acc = jnp.zeros((L,), jnp.float32)
                    for f in range(DW // L):
                        a = plsc.bitcast(
                            r0[row + edge, pl.ds(f * L, L)], jnp.bfloat16)
                        b = plsc.bitcast(
                            r1[row + edge, pl.ds(f * L, L)], jnp.bfloat16)
                        p = plsc.bitcast(a * b, jnp.int32)
                        hi = plsc.bitcast(p & hi_mask, jnp.float32)
                        lo = plsc.bitcast(p << 16, jnp.float32)
                        acc = acc + hi + lo
                    tmp[pl.ds(base + edge * L, L)] = plsc.cumsum(acc)

                tot = plsc.load_gather(tmp, [base + lane15])
                out_v[pl.ds(c * CHUNK + g * L, L)] = (
                    1.0 / (1.0 + jnp.exp(-tot)))

        start(0, CHUNK, 0, s0a, s1a)

        def pair_body(p, carry):
            c = p * 2
            start(c + 1, CHUNK, 1, s0b, s1b)
            wait(c, CHUNK, 0, s0a, s1a)
            compute(c, CHUNK, 0)

            @pl.when(p < N_FULL // 2 - 1)
            def _():
                start(c + 2, CHUNK, 0, s0a, s1a)

            wait(c + 1, CHUNK, 1, s0b, s1b)
            compute(c + 1, CHUNK, 1)
            return carry

        lax.fori_loop(0, N_FULL // 2, pair_body, 0, unroll=False)

        start(N_FULL, TAIL, 0, s0a, s1a)
        wait(N_FULL, TAIL, 0, s0a, s1a)
        compute(N_FULL, TAIL, 0)

        pltpu.sync_copy(out_v, out_hbm.at[pl.ds(wid * W_EDGES, W_EDGES)])

    return k(z, e)


def kernel(z, e):
    return _edge_dot_sigmoid(z, e.astype(jnp.int32))


# speedup vs baseline: 1.8958x; 1.8958x over previous
import functools

import jax
import jax.numpy as jnp
from jax import lax
from jax.experimental import pallas as pl
from jax.experimental.pallas import tpu as pltpu
from jax.experimental.pallas import tpu_sc as plsc

NC = 2
NS = 16
NW = NC * NS
L = 16

E_TOTAL = 320000
W_EDGES = E_TOTAL // NW
CHUNK = 128
N_FULL = W_EDGES // CHUNK
TAIL = W_EDGES - N_FULL * CHUNK
D = 128
DW = D // 2
V = 10000
ROWS_PER_SUB = V // NS
BLK = 125


def _edge_dot_sigmoid(z, e):
    mesh = plsc.VectorSubcoreMesh(core_axis_name="c", subcore_axis_name="s")

    @functools.partial(
        pl.kernel,
        mesh=mesh,
        out_type=jax.ShapeDtypeStruct((E_TOTAL,), jnp.float32),
        compiler_params=pltpu.CompilerParams(
            needs_layout_passes=False, use_tc_tiling_on_sc=False),
        scratch_types=[
            pltpu.VMEM((W_EDGES,), jnp.int32),
            pltpu.VMEM((W_EDGES,), jnp.int32),
            pltpu.VMEM((2, CHUNK, DW), jnp.int32),
            pltpu.VMEM((2, CHUNK, DW), jnp.int32),
            pltpu.VMEM((CHUNK * L,), jnp.float32),
            pltpu.VMEM((W_EDGES,), jnp.float32),
            pltpu.VMEM((BLK, D), jnp.float32),
            pltpu.VMEM((BLK, DW), jnp.int32),
            pltpu.VMEM_SHARED((V, DW), jnp.int32),
            pltpu.SemaphoreType.DMA,
            pltpu.SemaphoreType.DMA,
            pltpu.SemaphoreType.DMA,
            pltpu.SemaphoreType.DMA,
        ],
    )
    def k(z_hbm, e_hbm, out_hbm,
          idx0, idx1, rows0, rows1, tmp, out_v, stage_f, pack_b, z_sh,
          s0a, s1a, s0b, s1b):
        wid = lax.axis_index("s") * NC + lax.axis_index("c")
        sid = lax.axis_index("s")

        pltpu.sync_copy(e_hbm.at[0, pl.ds(wid * W_EDGES, W_EDGES)], idx0)
        pltpu.sync_copy(e_hbm.at[1, pl.ds(wid * W_EDGES, W_EDGES)], idx1)
        r0s = sid * ROWS_PER_SUB

        def stage_block(blk, carry):
            row0 = r0s + blk * BLK
            pltpu.sync_copy(z_hbm.at[pl.ds(row0, BLK)], stage_f)

            @plsc.parallel_loop(0, BLK, unroll=2)
            def pack_row(r):
                for w in range(D // (2 * L)):
                    a = stage_f[r, pl.ds(w * 2 * L, L)]
                    b = stage_f[r, pl.ds(w * 2 * L + L, L)]
                    pk = plsc.pack(a, b, format=plsc.PackFormat.INTERLEAVED)
                    pack_b[r, pl.ds(w * L, L)] = plsc.bitcast(pk, jnp.int32)

            pltpu.sync_copy(pack_b, z_sh.at[pl.ds(row0, BLK)])
            return carry

        lax.fori_loop(0, ROWS_PER_SUB // BLK, stage_block, 0, unroll=False)
        plsc.subcore_barrier()

        z_sh_i = z_sh

        def start(c, n, buf, sem0, sem1):
            pltpu.async_copy(z_sh_i.at[idx0.at[pl.ds(c * CHUNK, n)]],
                             rows0.at[buf, pl.ds(0, n)], sem0)
            pltpu.async_copy(z_sh_i.at[idx1.at[pl.ds(c * CHUNK, n)]],
                             rows1.at[buf, pl.ds(0, n)], sem1)

        def wait(c, n, buf, sem0, sem1):
            pltpu.make_async_copy(
                z_sh_i.at[idx0.at[pl.ds(c * CHUNK, n)]],
                rows0.at[buf, pl.ds(0, n)], sem0).wait()
            pltpu.make_async_copy(
                z_sh_i.at[idx1.at[pl.ds(c * CHUNK, n)]],
                rows1.at[buf, pl.ds(0, n)], sem1).wait()

        def compute(c, n, buf):
            r0 = rows0.at[buf]
            r1 = rows1.at[buf]
            hi_mask = jnp.full((L,), -65536, jnp.int32)
            lane = lax.iota(jnp.int32, L) * L

            @plsc.parallel_loop(0, n // L, unroll=2 if n > L else 1)
            def group_body(g):
                row = g * L
                base = g * (L * L)
                for edge in range(L):
                    acc = jnp.zeros((L,), jnp.float32)
                    for f in range(DW // L):
                        a = plsc.bitcast(
                            r0[row + edge, pl.ds(f * L, L)], jnp.bfloat16)
                        b = plsc.bitcast(
                            r1[row + edge, pl.ds(f * L, L)], jnp.bfloat16)
                        p = plsc.bitcast(a * b, jnp.int32)
                        hi = plsc.bitcast(p & hi_mask, jnp.float32)
                        lo = plsc.bitcast(p << 16, jnp.float32)
                        acc = acc + hi + lo
                    tmp[pl.ds(base + edge * L, L)] = acc

                tot = jnp.zeros((L,), jnp.float32)
                for l in range(L):
                    tot = tot + plsc.load_gather(tmp, [lane + (base + l)])
                out_v[pl.ds(c * CHUNK + g * L, L)] = (
                    1.0 / (1.0 + jnp.exp(-tot)))

        start(0, CHUNK, 0, s0a, s1a)

        def pair_body(p, carry):
            c = p * 2
            start(c + 1, CHUNK, 1, s0b, s1b)
            wait(c, CHUNK, 0, s0a, s1a)
            compute(c, CHUNK, 0)

            @pl.when(p < N_FULL // 2 - 1)
            def _():
                start(c + 2, CHUNK, 0, s0a, s1a)

            wait(c + 1, CHUNK, 1, s0b, s1b)
            compute(c + 1, CHUNK, 1)
            return carry

        lax.fori_loop(0, N_FULL // 2, pair_body, 0, unroll=False)

        start(N_FULL, TAIL, 0, s0a, s1a)
        wait(N_FULL, TAIL, 0, s0a, s1a)
        compute(N_FULL, TAIL, 0)

        pltpu.sync_copy(out_v, out_hbm.at[pl.ds(wid * W_EDGES, W_EDGES)])

    return k(z, e)


def kernel(z, e):
    return _edge_dot_sigmoid(z, e.astype(jnp.int32))
